# baseline (speedup 1.0000x reference)
import functools

import jax
import jax.numpy as jnp
from jax import lax
from jax.experimental import pallas as pl
from jax.experimental.pallas import tpu as pltpu
from jax.experimental.pallas import tpu_sc as plsc

_N0 = 2500.0
_LANES = 16


def _preprocess_sc(xf, patt, *, num_cores, num_subcores, interpret=False):
    num_workers = num_cores * num_subcores
    total_in = xf.shape[0]
    m = patt.shape[0]
    total_out = total_in // 2
    in_per_w = total_in // num_workers
    out_per_w = total_out // num_workers
    vecs_per_w = out_per_w // _LANES
    assert m & (m - 1) == 0, "Patt length must be a power of two"

    mesh = plsc.VectorSubcoreMesh(
        core_axis_name="c", subcore_axis_name="s",
        num_cores=num_cores, num_subcores=num_subcores,
    )

    @functools.partial(
        pl.kernel,
        out_type=jax.ShapeDtypeStruct((total_out,), jnp.float32),
        mesh=mesh,
        scratch_types=[
            pltpu.VMEM((in_per_w,), jnp.float32),
            pltpu.VMEM((m,), jnp.float32),
            pltpu.VMEM((out_per_w,), jnp.float32),
        ],
        compiler_params=pltpu.CompilerParams(needs_layout_passes=False),
        interpret=interpret,
    )
    def run(x_hbm, patt_hbm, out_hbm, x_v, patt_v, out_v):
        wid = lax.axis_index("s") * num_cores + lax.axis_index("c")
        pltpu.sync_copy(x_hbm.at[pl.ds(wid * in_per_w, in_per_w)], x_v)
        pltpu.sync_copy(patt_hbm, patt_v)

        even_iota = 2 * lax.iota(jnp.int32, _LANES)
        odd_iota = even_iota + 1
        scale = jnp.float32(2.0 / _N0)

        @plsc.parallel_loop(0, vecs_per_w, 1, unroll=8)
        def body(v):
            base = v * 32
            even = plsc.load_gather(x_v, [base + even_iota])
            odd = plsc.load_gather(x_v, [base + odd_iota])
            pm = lax.bitwise_and(v * _LANES, m - 1)
            p = patt_v[pl.ds(pm, _LANES)]
            out_v[pl.ds(v * _LANES, _LANES)] = (even - odd) * scale - p

        pltpu.sync_copy(out_v, out_hbm.at[pl.ds(wid * out_per_w, out_per_w)])

    return run(xf, patt)


def kernel(x, Patt, b, c, h, w):
    bs, cs, two_m = x.shape
    m = Patt.shape[0]
    xf = jnp.reshape(x, (bs * cs * two_m,))
    info = plsc.get_sparse_core_info()
    out = _preprocess_sc(xf, Patt.astype(jnp.float32),
                         num_cores=info.num_cores,
                         num_subcores=info.num_subcores)
    return jnp.reshape(out, (bs, cs, m))
